# Initial kernel scaffold; baseline (speedup 1.0000x reference)
import jax
import jax.numpy as jnp
from jax.experimental import pallas as pl


def kernel(x, edge_index, pairs, W1, a1_src, a1_dst, W2, a2_src, a2_dst, Wc, bc):
    raise NotImplementedError("write your pallas kernel here")


# speedup vs baseline: 14.8764x; 14.8764x over previous
import functools
import jax
import jax.numpy as jnp
from jax import lax
from jax.experimental import pallas as pl
from jax.experimental.pallas import tpu as pltpu
from jax.experimental.pallas import tpu_sc as plsc

LANES = 16
NTILES = 32
K = 128


def _round_up(v, m):
    return (v + m - 1) // m * m


def _mm_first(x_pad, W, A, BN):
    NP, D = x_pad.shape
    H = W.shape[1]

    def body(x_ref, w_ref, a_ref, h_ref, e_ref):
        h = jnp.dot(x_ref[...], w_ref[...], preferred_element_type=jnp.float32)
        h_ref[...] = h
        e_ref[...] = jnp.dot(h, a_ref[...], preferred_element_type=jnp.float32)

    return pl.pallas_call(
        body,
        grid=(NP // BN,),
        in_specs=[
            pl.BlockSpec((BN, D), lambda i: (i, 0)),
            pl.BlockSpec((D, H), lambda i: (0, 0)),
            pl.BlockSpec((H, 2), lambda i: (0, 0)),
        ],
        out_specs=[
            pl.BlockSpec((BN, H), lambda i: (i, 0)),
            pl.BlockSpec((BN, 2), lambda i: (i, 0)),
        ],
        out_shape=[
            jax.ShapeDtypeStruct((NP, H), jnp.float32),
            jax.ShapeDtypeStruct((NP, 2), jnp.float32),
        ],
    )(x_pad, W, A)


def _mm_relu_sum(o_flat, W, A, NP, BN):
    H = W.shape[1]
    D = o_flat.shape[1]

    def body(x0_ref, x1_ref, w_ref, a_ref, h_ref, e_ref):
        xin = jnp.maximum(x0_ref[...] + x1_ref[...], 0.0)
        h = jnp.dot(xin, w_ref[...], preferred_element_type=jnp.float32)
        h_ref[...] = h
        e_ref[...] = jnp.dot(h, a_ref[...], preferred_element_type=jnp.float32)

    nblk = NP // BN
    return pl.pallas_call(
        body,
        grid=(nblk,),
        in_specs=[
            pl.BlockSpec((BN, D), lambda i: (i, 0)),
            pl.BlockSpec((BN, D), lambda i, _n=nblk: (i + _n, 0)),
            pl.BlockSpec((D, H), lambda i: (0, 0)),
            pl.BlockSpec((H, 2), lambda i: (0, 0)),
        ],
        out_specs=[
            pl.BlockSpec((BN, H), lambda i: (i, 0)),
            pl.BlockSpec((BN, 2), lambda i: (i, 0)),
        ],
        out_shape=[
            jax.ShapeDtypeStruct((NP, H), jnp.float32),
            jax.ShapeDtypeStruct((NP, 2), jnp.float32),
        ],
    )(o_flat, o_flat, W, A)


def _make_sc_layer(NP, H, ET):
    NS = NP // LANES
    SEGS = ET // K
    mesh = plsc.VectorSubcoreMesh(core_axis_name="c", subcore_axis_name="s")

    @functools.partial(
        pl.kernel,
        out_type=jax.ShapeDtypeStruct((2 * NP, H), jnp.float32),
        mesh=mesh,
        scratch_types=[
            pltpu.VMEM_SHARED((NP, H), jnp.float32),
            pltpu.VMEM_SHARED((NP,), jnp.float32),
            pltpu.VMEM_SHARED((NP,), jnp.float32),
            pltpu.VMEM_SHARED((NP,), jnp.float32),
            pltpu.VMEM((NP,), jnp.float32),
            pltpu.VMEM((ET,), jnp.float32),
            pltpu.VMEM((K, H), jnp.float32),
            pltpu.VMEM((K,), jnp.int32),
            pltpu.VMEM((K,), jnp.int32),
            pltpu.VMEM((K,), jnp.float32),
            pltpu.VMEM((K,), jnp.float32),
            pltpu.VMEM((K,), jnp.float32),
            pltpu.VMEM((K,), jnp.float32),
            pltpu.VMEM((16,), jnp.float32),
            pltpu.SemaphoreType.DMA,
            pltpu.SemaphoreType.DMA,
        ],
    )
    def sc_layer(h_hbm, es_hbm, ed_hbm, src_hbm, dst_hbm, m_hbm, out_hbm,
                 acc_sh, denom_sh, es_sh, ed_sh, stage_v, ex_keep, rows_v,
                 si_v, di_v, ga_v, gb_v, ex_v, attn_v, m_v, sem0, sem1):
        cid = lax.axis_index("c")
        sid = lax.axis_index("s")

        @pl.when(sid == 0)
        def _():
            pltpu.sync_copy(es_hbm, stage_v)
            pltpu.sync_copy(stage_v, es_sh)

        @pl.when(sid == 1)
        def _():
            pltpu.sync_copy(ed_hbm, stage_v)
            pltpu.sync_copy(stage_v, ed_sh)

        zv = jnp.zeros((LANES,), jnp.float32)

        def zbody(r, _):
            for j in range(H // LANES):
                rows_v[r, pl.ds(LANES * j, LANES)] = zv
            return 0
        lax.fori_loop(0, K, zbody, 0)

        @pl.when(sid == 2)
        def _():
            def dzbody(t, _):
                pltpu.sync_copy(rows_v.at[0], denom_sh.at[pl.ds(t * H, H)])
                return 0
            lax.fori_loop(0, NP // H, dzbody, 0)

        for t in range(NS // K):
            pltpu.sync_copy(rows_v, acc_sh.at[pl.ds(sid * NS + t * K, K)])

        pltpu.sync_copy(m_hbm, m_v)
        mvec = m_v[...]
        plsc.subcore_barrier()

        own_chunk = cid * 16 + sid
        other_chunk = (1 - cid) * 16 + sid

        for chunk_base, keep in ((other_chunk * ET, False), (own_chunk * ET, True)):
            def abody(seg, _, chunk_base=chunk_base, keep=keep):
                off = chunk_base + seg * K
                pltpu.sync_copy(src_hbm.at[pl.ds(off, K)], si_v)
                pltpu.sync_copy(dst_hbm.at[pl.ds(off, K)], di_v)
                cpa = pltpu.async_copy(es_sh.at[si_v], ga_v, sem0)
                cpb = pltpu.async_copy(ed_sh.at[di_v], gb_v, sem1)
                cpa.wait()
                cpb.wait()
                for i in range(K // LANES):
                    sl = pl.ds(LANES * i, LANES)
                    e = ga_v[sl] + gb_v[sl]
                    e = jnp.where(e > 0, e, 0.2 * e)
                    ex = jnp.exp(e - mvec)
                    ex_v[sl] = ex
                    if keep:
                        ex_keep[pl.ds(seg * K + LANES * i, LANES)] = ex
                pltpu.sync_copy(ex_v, denom_sh.at[di_v], add=True)
                return 0
            lax.fori_loop(0, SEGS, abody, 0)

        plsc.subcore_barrier()

        own_base = own_chunk * ET

        def bbody(seg, _):
            off = own_base + seg * K
            pltpu.sync_copy(src_hbm.at[pl.ds(off, K)], si_v)
            pltpu.sync_copy(dst_hbm.at[pl.ds(off, K)], di_v)
            rows_cp = pltpu.async_copy(h_hbm.at[si_v], rows_v, sem0)
            pltpu.sync_copy(denom_sh.at[di_v], ga_v)
            for i in range(K // LANES):
                sl = pl.ds(LANES * i, LANES)
                attn_v[sl] = ex_keep[pl.ds(seg * K + LANES * i, LANES)] / (ga_v[sl] + 1e-9)
            rows_cp.wait()

            def sbody(g, _):
                av = attn_v[pl.ds(g * LANES, LANES)]
                for rl in range(LANES):
                    r = g * LANES + rl
                    a = av[rl]
                    for j in range(H // LANES):
                        sl = pl.ds(LANES * j, LANES)
                        rows_v[r, sl] = rows_v[r, sl] * a
                return 0
            lax.fori_loop(0, K // LANES, sbody, 0)
            pltpu.sync_copy(rows_v, acc_sh.at[di_v], add=True)
            return 0
        lax.fori_loop(0, SEGS, bbody, 0)

        plsc.subcore_barrier()

        for t in range(NS // K):
            row0 = sid * NS + t * K
            pltpu.sync_copy(acc_sh.at[pl.ds(row0, K)],
                            out_hbm.at[pl.ds(cid * NP + row0, K)])

    return sc_layer


def _make_sc_pairs(NP, H, P):
    PT = P // NTILES
    mesh = plsc.VectorSubcoreMesh(core_axis_name="c", subcore_axis_name="s")

    @functools.partial(
        pl.kernel,
        out_type=[jax.ShapeDtypeStruct((P, H), jnp.float32),
                  jax.ShapeDtypeStruct((P, H), jnp.float32)],
        mesh=mesh,
        scratch_types=[
            pltpu.VMEM((K, H), jnp.float32),
            pltpu.VMEM((K, H), jnp.float32),
            pltpu.VMEM((K,), jnp.int32),
            pltpu.VMEM((K,), jnp.int32),
            pltpu.SemaphoreType.DMA,
            pltpu.SemaphoreType.DMA,
        ],
    )
    def sc_pairs(hh_hbm, ps_hbm, pd_hbm, outa_hbm, outb_hbm,
                 rs, rd, ps_v, pd_v, s0, s1):
        cid = lax.axis_index("c")
        sid = lax.axis_index("s")
        wid = sid * 2 + cid
        base = wid * PT

        def seg_body(seg, _):
            off = base + seg * K
            pltpu.sync_copy(ps_hbm.at[pl.ds(off, K)], ps_v)
            pltpu.sync_copy(pd_hbm.at[pl.ds(off, K)], pd_v)
            c0 = pltpu.async_copy(hh_hbm.at[ps_v], rs, s0)
            c1 = pltpu.async_copy(hh_hbm.at[pd_v], rd, s1)
            c0.wait()
            pltpu.sync_copy(rs, outa_hbm.at[pl.ds(off, K)])
            c1.wait()
            pltpu.sync_copy(rd, outb_hbm.at[pl.ds(off, K)])
            return 0
        lax.fori_loop(0, PT // K, seg_body, 0)

    return sc_pairs


def _relu_sum(o_flat, NP, BN):
    H = o_flat.shape[1]

    def body(x0_ref, x1_ref, o_ref):
        o_ref[...] = jnp.maximum(x0_ref[...] + x1_ref[...], 0.0)

    nblk = NP // BN
    return pl.pallas_call(
        body,
        grid=(nblk,),
        in_specs=[
            pl.BlockSpec((BN, H), lambda i: (i, 0)),
            pl.BlockSpec((BN, H), lambda i, _n=nblk: (i + _n, 0)),
        ],
        out_specs=pl.BlockSpec((BN, H), lambda i: (i, 0)),
        out_shape=jax.ShapeDtypeStruct((NP, H), jnp.float32),
    )(o_flat, o_flat)


def _final_tc(hpa, hpb, Wc, bc, BP):
    P, H = hpa.shape
    Wc1 = Wc[:H]
    Wc2 = Wc[H:]
    bc2 = bc.reshape(1, 2)

    def body(a_ref, b_ref, w1_ref, w2_ref, bc_ref, o_ref):
        z = (jnp.dot(a_ref[...], w1_ref[...], preferred_element_type=jnp.float32)
             + jnp.dot(b_ref[...], w2_ref[...], preferred_element_type=jnp.float32)
             + bc_ref[...])
        o_ref[...] = 1.0 / (1.0 + jnp.exp(-z))

    return pl.pallas_call(
        body,
        grid=(P // BP,),
        in_specs=[
            pl.BlockSpec((BP, H), lambda i: (i, 0)),
            pl.BlockSpec((BP, H), lambda i: (i, 0)),
            pl.BlockSpec((H, 2), lambda i: (0, 0)),
            pl.BlockSpec((H, 2), lambda i: (0, 0)),
            pl.BlockSpec((1, 2), lambda i: (0, 0)),
        ],
        out_specs=pl.BlockSpec((BP, 2), lambda i: (i, 0)),
        out_shape=jax.ShapeDtypeStruct((P, 2), jnp.float32),
    )(hpa, hpb, Wc1, Wc2, bc2)


def kernel(x, edge_index, pairs, W1, a1_src, a1_dst, W2, a2_src, a2_dst, Wc, bc):
    N, D = x.shape
    H = W1.shape[1]
    E = edge_index.shape[1]
    P = pairs.shape[0]

    NP = _round_up(N, LANES * K)
    EP = _round_up(E, NTILES * K)
    ET = EP // NTILES
    BN = 2048 if NP % 2048 == 0 else LANES * K

    x_pad = jnp.zeros((NP, D), jnp.float32).at[:N, :].set(x)
    pad_node = NP - 1
    src_pad = jnp.full((EP,), pad_node, jnp.int32).at[:E].set(edge_index[0])
    dst_pad = jnp.full((EP,), pad_node, jnp.int32).at[:E].set(edge_index[1])
    A1 = jnp.stack([a1_src, a1_dst], axis=1)
    A2 = jnp.stack([a2_src, a2_dst], axis=1)
    ps = pairs[:, 0].astype(jnp.int32)
    pd = pairs[:, 1].astype(jnp.int32)

    sc_layer = _make_sc_layer(NP, H, ET)
    sc_pairs = _make_sc_pairs(NP, H, P)

    def logit_bound(esd):
        m = jnp.max(esd[:, 0]) + jnp.max(esd[:, 1])
        m = jnp.where(m > 0, m, 0.2 * m)
        return jnp.full((16,), m, jnp.float32)

    h1, esd1 = _mm_first(x_pad, W1, A1, BN)
    o1 = sc_layer(h1, esd1[:, 0], esd1[:, 1], src_pad, dst_pad, logit_bound(esd1))
    h2, esd2 = _mm_relu_sum(o1, W2, A2, NP, BN)
    o2 = sc_layer(h2, esd2[:, 0], esd2[:, 1], src_pad, dst_pad, logit_bound(esd2))
    hh = _relu_sum(o2, NP, BN)
    hpa, hpb = sc_pairs(hh, ps, pd)
    return _final_tc(hpa, hpb, Wc, bc, 2048)


# speedup vs baseline: 14.9233x; 1.0032x over previous
import functools
import jax
import jax.numpy as jnp
from jax import lax
from jax.experimental import pallas as pl
from jax.experimental.pallas import tpu as pltpu
from jax.experimental.pallas import tpu_sc as plsc

LANES = 16
NTILES = 32
K = 128


def _round_up(v, m):
    return (v + m - 1) // m * m


def _mm_first(x_pad, W, A, BN):
    NP, D = x_pad.shape
    H = W.shape[1]

    def body(x_ref, w_ref, a_ref, h_ref, e_ref):
        h = jnp.dot(x_ref[...], w_ref[...], preferred_element_type=jnp.float32)
        h_ref[...] = h
        e_ref[...] = jnp.dot(h, a_ref[...], preferred_element_type=jnp.float32)

    return pl.pallas_call(
        body,
        grid=(NP // BN,),
        in_specs=[
            pl.BlockSpec((BN, D), lambda i: (i, 0)),
            pl.BlockSpec((D, H), lambda i: (0, 0)),
            pl.BlockSpec((H, 2), lambda i: (0, 0)),
        ],
        out_specs=[
            pl.BlockSpec((BN, H), lambda i: (i, 0)),
            pl.BlockSpec((BN, 2), lambda i: (i, 0)),
        ],
        out_shape=[
            jax.ShapeDtypeStruct((NP, H), jnp.float32),
            jax.ShapeDtypeStruct((NP, 2), jnp.float32),
        ],
    )(x_pad, W, A)


def _mm_relu_sum(o_flat, W, A, NP, BN):
    H = W.shape[1]
    D = o_flat.shape[1]

    def body(x0_ref, x1_ref, w_ref, a_ref, h_ref, e_ref):
        xin = jnp.maximum(x0_ref[...] + x1_ref[...], 0.0)
        h = jnp.dot(xin, w_ref[...], preferred_element_type=jnp.float32)
        h_ref[...] = h
        e_ref[...] = jnp.dot(h, a_ref[...], preferred_element_type=jnp.float32)

    nblk = NP // BN
    return pl.pallas_call(
        body,
        grid=(nblk,),
        in_specs=[
            pl.BlockSpec((BN, D), lambda i: (i, 0)),
            pl.BlockSpec((BN, D), lambda i, _n=nblk: (i + _n, 0)),
            pl.BlockSpec((D, H), lambda i: (0, 0)),
            pl.BlockSpec((H, 2), lambda i: (0, 0)),
        ],
        out_specs=[
            pl.BlockSpec((BN, H), lambda i: (i, 0)),
            pl.BlockSpec((BN, 2), lambda i: (i, 0)),
        ],
        out_shape=[
            jax.ShapeDtypeStruct((NP, H), jnp.float32),
            jax.ShapeDtypeStruct((NP, 2), jnp.float32),
        ],
    )(o_flat, o_flat, W, A)


GRP = 4


def _make_sc_layer(NP, H, ET):
    NS = NP // LANES
    SEGS = ET // K
    NGRP = SEGS // GRP
    mesh = plsc.VectorSubcoreMesh(core_axis_name="c", subcore_axis_name="s")

    @functools.partial(
        pl.kernel,
        out_type=jax.ShapeDtypeStruct((2 * NP, H), jnp.float32),
        mesh=mesh,
        scratch_types=[
            pltpu.VMEM_SHARED((NP, H), jnp.float32),
            pltpu.VMEM_SHARED((NP,), jnp.float32),
            pltpu.VMEM_SHARED((NP,), jnp.float32),
            pltpu.VMEM_SHARED((NP,), jnp.float32),
            [pltpu.VMEM((K, H), jnp.float32) for _ in range(2)],
            pltpu.VMEM((GRP * K,), jnp.int32),
            pltpu.VMEM((GRP * K,), jnp.int32),
            [pltpu.VMEM((K,), jnp.int32) for _ in range(GRP)],
            [pltpu.VMEM((K,), jnp.float32) for _ in range(GRP)],
            [pltpu.VMEM((K,), jnp.float32) for _ in range(GRP)],
            [pltpu.VMEM((K,), jnp.float32) for _ in range(GRP)],
            pltpu.VMEM((K,), jnp.float32),
            pltpu.VMEM((16,), jnp.float32),
            [pltpu.SemaphoreType.DMA for _ in range(4 * GRP)],
        ],
    )
    def sc_layer(h_hbm, es_hbm, ed_hbm, src_hbm, dst_hbm, m_hbm, out_hbm,
                 acc_sh, denom_sh, es_sh, ed_sh, rows, si_g, di_g,
                 di_scat, ga, gb, exv, attn_v, m_v, sems):
        cid = lax.axis_index("c")
        sid = lax.axis_index("s")

        @pl.when(sid == 0)
        def _():
            pltpu.sync_copy(es_hbm, es_sh)

        @pl.when(sid == 1)
        def _():
            pltpu.sync_copy(ed_hbm, ed_sh)

        zv = jnp.zeros((LANES,), jnp.float32)

        def zbody(r, _):
            for j in range(H // LANES):
                rows[0][r, pl.ds(LANES * j, LANES)] = zv
            return 0
        lax.fori_loop(0, K, zbody, 0)

        @pl.when(sid == 2)
        def _():
            def dzbody(t, _):
                pltpu.sync_copy(rows[0].at[0], denom_sh.at[pl.ds(t * H, H)])
                return 0
            lax.fori_loop(0, NP // H, dzbody, 0)

        for t in range(NS // K):
            pltpu.sync_copy(rows[0], acc_sh.at[pl.ds(sid * NS + t * K, K)])

        pltpu.sync_copy(m_hbm, m_v)
        mvec = m_v[...]
        plsc.subcore_barrier()

        own_chunk = cid * 16 + sid
        other_chunk = (1 - cid) * 16 + sid
        GK = GRP * K

        for chunk in (other_chunk, own_chunk):
            base = chunk * ET

            def agroup(g, _, base=base):
                gsl = pl.ds(base + g * GK, GK)
                ci = pltpu.async_copy(src_hbm.at[gsl], si_g, sems[3 * GRP])
                cj = pltpu.async_copy(dst_hbm.at[gsl], di_g, sems[3 * GRP + 1])
                ci.wait()
                cj.wait()
                cps = []
                for b in range(GRP):
                    sl = pl.ds(b * K, K)
                    cps.append((
                        pltpu.async_copy(es_sh.at[si_g.at[sl]], ga[b], sems[b]),
                        pltpu.async_copy(ed_sh.at[di_g.at[sl]], gb[b], sems[GRP + b]),
                    ))
                scats = []
                for b in range(GRP):
                    cps[b][0].wait()
                    cps[b][1].wait()
                    for i in range(K // LANES):
                        sl = pl.ds(LANES * i, LANES)
                        di_scat[b][sl] = di_g[pl.ds(b * K + LANES * i, LANES)]
                        e = ga[b][sl] + gb[b][sl]
                        e = jnp.where(e > 0, e, 0.2 * e)
                        exv[b][sl] = jnp.exp(e - mvec)
                    scats.append(pltpu.async_copy(
                        exv[b], denom_sh.at[di_scat[b]], sems[2 * GRP + b], add=True))
                for cp in scats:
                    cp.wait()
                return 0
            lax.fori_loop(0, NGRP, agroup, 0)

        plsc.subcore_barrier()

        GRPB = 2
        own_base = own_chunk * ET

        def bgroup(g, _):
            gsl = pl.ds(own_base + g * GRPB * K, GRPB * K)
            ci = pltpu.async_copy(src_hbm.at[gsl], si_g.at[pl.ds(0, GRPB * K)],
                                  sems[3 * GRP])
            cj = pltpu.async_copy(dst_hbm.at[gsl], di_g.at[pl.ds(0, GRPB * K)],
                                  sems[3 * GRP + 1])
            ci.wait()
            cj.wait()
            cps = []
            for b in range(GRPB):
                sl = pl.ds(b * K, K)
                cps.append((
                    pltpu.async_copy(h_hbm.at[si_g.at[sl]], rows[b], sems[b]),
                    pltpu.async_copy(es_sh.at[si_g.at[sl]], ga[b], sems[GRP + b]),
                    pltpu.async_copy(ed_sh.at[di_g.at[sl]], gb[b], sems[2 * GRP + b]),
                    pltpu.async_copy(denom_sh.at[di_g.at[sl]], exv[b], sems[3 * GRP + 2 + b]),
                ))
            scats = []
            for b in range(GRPB):
                cps[b][1].wait()
                cps[b][2].wait()
                cps[b][3].wait()
                for i in range(K // LANES):
                    sl = pl.ds(LANES * i, LANES)
                    di_scat[b][sl] = di_g[pl.ds(b * K + LANES * i, LANES)]
                    e = ga[b][sl] + gb[b][sl]
                    e = jnp.where(e > 0, e, 0.2 * e)
                    ex = jnp.exp(e - mvec)
                    attn_v[sl] = ex / (exv[b][sl] + 1e-9)
                cps[b][0].wait()

                def sbody(q, _, b=b):
                    av = attn_v[pl.ds(q * LANES, LANES)]
                    for rl in range(LANES):
                        r = q * LANES + rl
                        a = av[rl]
                        for j in range(H // LANES):
                            sl = pl.ds(LANES * j, LANES)
                            rows[b][r, sl] = rows[b][r, sl] * a
                    return 0
                lax.fori_loop(0, K // LANES, sbody, 0)
                scats.append(pltpu.async_copy(
                    rows[b], acc_sh.at[di_scat[b]], sems[3 * GRP - 2 + b], add=True))
            for cp in scats:
                cp.wait()
            return 0
        lax.fori_loop(0, SEGS // GRPB, bgroup, 0)

        plsc.subcore_barrier()

        for t in range(NS // K):
            row0 = sid * NS + t * K
            pltpu.sync_copy(acc_sh.at[pl.ds(row0, K)],
                            out_hbm.at[pl.ds(cid * NP + row0, K)])

    return sc_layer


def _make_sc_pairs(NP, H, P):
    PT = P // NTILES
    mesh = plsc.VectorSubcoreMesh(core_axis_name="c", subcore_axis_name="s")

    @functools.partial(
        pl.kernel,
        out_type=[jax.ShapeDtypeStruct((P, H), jnp.float32),
                  jax.ShapeDtypeStruct((P, H), jnp.float32)],
        mesh=mesh,
        scratch_types=[
            pltpu.VMEM((K, H), jnp.float32),
            pltpu.VMEM((K, H), jnp.float32),
            pltpu.VMEM((K,), jnp.int32),
            pltpu.VMEM((K,), jnp.int32),
            pltpu.SemaphoreType.DMA,
            pltpu.SemaphoreType.DMA,
        ],
    )
    def sc_pairs(hh_hbm, ps_hbm, pd_hbm, outa_hbm, outb_hbm,
                 rs, rd, ps_v, pd_v, s0, s1):
        cid = lax.axis_index("c")
        sid = lax.axis_index("s")
        wid = sid * 2 + cid
        base = wid * PT

        def seg_body(seg, _):
            off = base + seg * K
            pltpu.sync_copy(ps_hbm.at[pl.ds(off, K)], ps_v)
            pltpu.sync_copy(pd_hbm.at[pl.ds(off, K)], pd_v)
            c0 = pltpu.async_copy(hh_hbm.at[ps_v], rs, s0)
            c1 = pltpu.async_copy(hh_hbm.at[pd_v], rd, s1)
            c0.wait()
            pltpu.sync_copy(rs, outa_hbm.at[pl.ds(off, K)])
            c1.wait()
            pltpu.sync_copy(rd, outb_hbm.at[pl.ds(off, K)])
            return 0
        lax.fori_loop(0, PT // K, seg_body, 0)

    return sc_pairs


def _relu_sum(o_flat, NP, BN):
    H = o_flat.shape[1]

    def body(x0_ref, x1_ref, o_ref):
        o_ref[...] = jnp.maximum(x0_ref[...] + x1_ref[...], 0.0)

    nblk = NP // BN
    return pl.pallas_call(
        body,
        grid=(nblk,),
        in_specs=[
            pl.BlockSpec((BN, H), lambda i: (i, 0)),
            pl.BlockSpec((BN, H), lambda i, _n=nblk: (i + _n, 0)),
        ],
        out_specs=pl.BlockSpec((BN, H), lambda i: (i, 0)),
        out_shape=jax.ShapeDtypeStruct((NP, H), jnp.float32),
    )(o_flat, o_flat)


def _final_tc(hpa, hpb, Wc, bc, BP):
    P, H = hpa.shape
    Wc1 = Wc[:H]
    Wc2 = Wc[H:]
    bc2 = bc.reshape(1, 2)

    def body(a_ref, b_ref, w1_ref, w2_ref, bc_ref, o_ref):
        z = (jnp.dot(a_ref[...], w1_ref[...], preferred_element_type=jnp.float32)
             + jnp.dot(b_ref[...], w2_ref[...], preferred_element_type=jnp.float32)
             + bc_ref[...])
        o_ref[...] = 1.0 / (1.0 + jnp.exp(-z))

    return pl.pallas_call(
        body,
        grid=(P // BP,),
        in_specs=[
            pl.BlockSpec((BP, H), lambda i: (i, 0)),
            pl.BlockSpec((BP, H), lambda i: (i, 0)),
            pl.BlockSpec((H, 2), lambda i: (0, 0)),
            pl.BlockSpec((H, 2), lambda i: (0, 0)),
            pl.BlockSpec((1, 2), lambda i: (0, 0)),
        ],
        out_specs=pl.BlockSpec((BP, 2), lambda i: (i, 0)),
        out_shape=jax.ShapeDtypeStruct((P, 2), jnp.float32),
    )(hpa, hpb, Wc1, Wc2, bc2)


def kernel(x, edge_index, pairs, W1, a1_src, a1_dst, W2, a2_src, a2_dst, Wc, bc):
    N, D = x.shape
    H = W1.shape[1]
    E = edge_index.shape[1]
    P = pairs.shape[0]

    NP = _round_up(N, LANES * K)
    EP = _round_up(E, NTILES * K * GRP)
    ET = EP // NTILES
    BN = 2048 if NP % 2048 == 0 else LANES * K

    x_pad = jnp.zeros((NP, D), jnp.float32).at[:N, :].set(x)
    pad_node = NP - 1
    src_pad = jnp.full((EP,), pad_node, jnp.int32).at[:E].set(edge_index[0])
    dst_pad = jnp.full((EP,), pad_node, jnp.int32).at[:E].set(edge_index[1])
    A1 = jnp.stack([a1_src, a1_dst], axis=1)
    A2 = jnp.stack([a2_src, a2_dst], axis=1)
    ps = pairs[:, 0].astype(jnp.int32)
    pd = pairs[:, 1].astype(jnp.int32)

    sc_layer = _make_sc_layer(NP, H, ET)
    sc_pairs = _make_sc_pairs(NP, H, P)

    def logit_bound(esd):
        m = jnp.max(esd[:, 0]) + jnp.max(esd[:, 1])
        m = jnp.where(m > 0, m, 0.2 * m)
        return jnp.full((16,), m, jnp.float32)

    h1, esd1 = _mm_first(x_pad, W1, A1, BN)
    o1 = sc_layer(h1, esd1[:, 0], esd1[:, 1], src_pad, dst_pad, logit_bound(esd1))
    h2, esd2 = _mm_relu_sum(o1, W2, A2, NP, BN)
    o2 = sc_layer(h2, esd2[:, 0], esd2[:, 1], src_pad, dst_pad, logit_bound(esd2))
    hh = _relu_sum(o2, NP, BN)
    hpa, hpb = sc_pairs(hh, ps, pd)
    return _final_tc(hpa, hpb, Wc, bc, 2048)


# speedup vs baseline: 14.9270x; 1.0002x over previous
import functools
import jax
import jax.numpy as jnp
from jax import lax
from jax.experimental import pallas as pl
from jax.experimental.pallas import tpu as pltpu
from jax.experimental.pallas import tpu_sc as plsc

LANES = 16
NTILES = 32
K = 128


def _round_up(v, m):
    return (v + m - 1) // m * m


def _mm_first(x_pad, W, A, BN):
    NP, D = x_pad.shape
    H = W.shape[1]

    def body(x_ref, w_ref, a_ref, h_ref, e_ref):
        h = jnp.dot(x_ref[...], w_ref[...], preferred_element_type=jnp.float32)
        h_ref[...] = h
        e_ref[...] = jnp.dot(h, a_ref[...], preferred_element_type=jnp.float32)

    return pl.pallas_call(
        body,
        grid=(NP // BN,),
        in_specs=[
            pl.BlockSpec((BN, D), lambda i: (i, 0)),
            pl.BlockSpec((D, H), lambda i: (0, 0)),
            pl.BlockSpec((H, 2), lambda i: (0, 0)),
        ],
        out_specs=[
            pl.BlockSpec((BN, H), lambda i: (i, 0)),
            pl.BlockSpec((BN, 2), lambda i: (i, 0)),
        ],
        out_shape=[
            jax.ShapeDtypeStruct((NP, H), jnp.float32),
            jax.ShapeDtypeStruct((NP, 2), jnp.float32),
        ],
    )(x_pad, W, A)


def _mm_relu_sum(o_flat, W, A, NP, BN):
    H = W.shape[1]
    D = o_flat.shape[1]

    def body(x0_ref, x1_ref, w_ref, a_ref, h_ref, e_ref):
        xin = jnp.maximum(x0_ref[...] + x1_ref[...], 0.0)
        h = jnp.dot(xin, w_ref[...], preferred_element_type=jnp.float32)
        h_ref[...] = h
        e_ref[...] = jnp.dot(h, a_ref[...], preferred_element_type=jnp.float32)

    nblk = NP // BN
    return pl.pallas_call(
        body,
        grid=(nblk,),
        in_specs=[
            pl.BlockSpec((BN, D), lambda i: (i, 0)),
            pl.BlockSpec((BN, D), lambda i, _n=nblk: (i + _n, 0)),
            pl.BlockSpec((D, H), lambda i: (0, 0)),
            pl.BlockSpec((H, 2), lambda i: (0, 0)),
        ],
        out_specs=[
            pl.BlockSpec((BN, H), lambda i: (i, 0)),
            pl.BlockSpec((BN, 2), lambda i: (i, 0)),
        ],
        out_shape=[
            jax.ShapeDtypeStruct((NP, H), jnp.float32),
            jax.ShapeDtypeStruct((NP, 2), jnp.float32),
        ],
    )(o_flat, o_flat, W, A)


GRP = 4


def _make_sc_layer(NP, H, ET):
    NS = NP // LANES
    SEGS = ET // K
    NGRP = SEGS // GRP
    mesh = plsc.VectorSubcoreMesh(core_axis_name="c", subcore_axis_name="s")

    @functools.partial(
        pl.kernel,
        out_type=jax.ShapeDtypeStruct((2 * NP, H), jnp.float32),
        mesh=mesh,
        scratch_types=[
            pltpu.VMEM_SHARED((NP, H), jnp.float32),
            pltpu.VMEM_SHARED((NP,), jnp.float32),
            pltpu.VMEM_SHARED((NP,), jnp.float32),
            pltpu.VMEM_SHARED((NP,), jnp.float32),
            [pltpu.VMEM((K, H), jnp.float32) for _ in range(2)],
            pltpu.VMEM((GRP * K,), jnp.int32),
            pltpu.VMEM((GRP * K,), jnp.int32),
            [pltpu.VMEM((K,), jnp.int32) for _ in range(GRP)],
            [pltpu.VMEM((K,), jnp.float32) for _ in range(GRP)],
            [pltpu.VMEM((K,), jnp.float32) for _ in range(GRP)],
            [pltpu.VMEM((K,), jnp.float32) for _ in range(GRP)],
            pltpu.VMEM((K,), jnp.float32),
            pltpu.VMEM((16,), jnp.float32),
            [pltpu.SemaphoreType.DMA for _ in range(4 * GRP)],
        ],
    )
    def sc_layer(h_hbm, es_hbm, ed_hbm, src_hbm, dst_hbm, m_hbm, out_hbm,
                 acc_sh, denom_sh, es_sh, ed_sh, rows, si_g, di_g,
                 di_scat, ga, gb, exv, attn_v, m_v, sems):
        cid = lax.axis_index("c")
        sid = lax.axis_index("s")

        @pl.when(sid == 0)
        def _():
            pltpu.sync_copy(es_hbm, es_sh)

        @pl.when(sid == 1)
        def _():
            pltpu.sync_copy(ed_hbm, ed_sh)

        zv = jnp.zeros((LANES,), jnp.float32)

        def zbody(r, _):
            for j in range(H // LANES):
                rows[0][r, pl.ds(LANES * j, LANES)] = zv
            return 0
        lax.fori_loop(0, K, zbody, 0)

        @pl.when(sid == 2)
        def _():
            def dzbody(t, _):
                pltpu.sync_copy(rows[0].at[0], denom_sh.at[pl.ds(t * H, H)])
                return 0
            lax.fori_loop(0, NP // H, dzbody, 0)

        for t in range(NS // K):
            pltpu.sync_copy(rows[0], acc_sh.at[pl.ds(sid * NS + t * K, K)])

        pltpu.sync_copy(m_hbm, m_v)
        mvec = m_v[...]
        plsc.subcore_barrier()

        own_chunk = cid * 16 + sid
        other_chunk = (1 - cid) * 16 + sid
        GK = GRP * K

        for chunk in (other_chunk, own_chunk):
            base = chunk * ET
            scope = jax.named_scope("phaseA")
            scope.__enter__()

            def agroup(g, _, base=base):
                gsl = pl.ds(base + g * GK, GK)
                ci = pltpu.async_copy(src_hbm.at[gsl], si_g, sems[3 * GRP])
                cj = pltpu.async_copy(dst_hbm.at[gsl], di_g, sems[3 * GRP + 1])
                ci.wait()
                cj.wait()
                cps = []
                for b in range(GRP):
                    sl = pl.ds(b * K, K)
                    cps.append((
                        pltpu.async_copy(es_sh.at[si_g.at[sl]], ga[b], sems[b]),
                        pltpu.async_copy(ed_sh.at[di_g.at[sl]], gb[b], sems[GRP + b]),
                    ))
                scats = []
                for b in range(GRP):
                    cps[b][0].wait()
                    cps[b][1].wait()
                    for i in range(K // LANES):
                        sl = pl.ds(LANES * i, LANES)
                        di_scat[b][sl] = di_g[pl.ds(b * K + LANES * i, LANES)]
                        e = ga[b][sl] + gb[b][sl]
                        e = jnp.where(e > 0, e, 0.2 * e)
                        exv[b][sl] = jnp.exp(e - mvec)
                    scats.append(pltpu.async_copy(
                        exv[b], denom_sh.at[di_scat[b]], sems[2 * GRP + b], add=True))
                for cp in scats:
                    cp.wait()
                return 0
            lax.fori_loop(0, NGRP, agroup, 0)
            scope.__exit__(None, None, None)

        plsc.subcore_barrier()

        GRPB = 2
        own_base = own_chunk * ET

        def bgroup(g, _):
            gsl = pl.ds(own_base + g * GRPB * K, GRPB * K)
            ci = pltpu.async_copy(src_hbm.at[gsl], si_g.at[pl.ds(0, GRPB * K)],
                                  sems[3 * GRP])
            cj = pltpu.async_copy(dst_hbm.at[gsl], di_g.at[pl.ds(0, GRPB * K)],
                                  sems[3 * GRP + 1])
            ci.wait()
            cj.wait()
            cps = []
            for b in range(GRPB):
                sl = pl.ds(b * K, K)
                cps.append((
                    pltpu.async_copy(h_hbm.at[si_g.at[sl]], rows[b], sems[b]),
                    pltpu.async_copy(es_sh.at[si_g.at[sl]], ga[b], sems[GRP + b]),
                    pltpu.async_copy(ed_sh.at[di_g.at[sl]], gb[b], sems[2 * GRP + b]),
                    pltpu.async_copy(denom_sh.at[di_g.at[sl]], exv[b], sems[3 * GRP + 2 + b]),
                ))
            scats = []
            for b in range(GRPB):
                cps[b][1].wait()
                cps[b][2].wait()
                cps[b][3].wait()
                for i in range(K // LANES):
                    sl = pl.ds(LANES * i, LANES)
                    di_scat[b][sl] = di_g[pl.ds(b * K + LANES * i, LANES)]
                    e = ga[b][sl] + gb[b][sl]
                    e = jnp.where(e > 0, e, 0.2 * e)
                    ex = jnp.exp(e - mvec)
                    attn_v[sl] = ex / (exv[b][sl] + 1e-9)
                cps[b][0].wait()

                def sbody(q, _, b=b):
                    av = attn_v[pl.ds(q * LANES, LANES)]
                    for rl in range(LANES):
                        r = q * LANES + rl
                        a = av[rl]
                        for j in range(H // LANES):
                            sl = pl.ds(LANES * j, LANES)
                            rows[b][r, sl] = rows[b][r, sl] * a
                    return 0
                lax.fori_loop(0, K // LANES, sbody, 0)
                scats.append(pltpu.async_copy(
                    rows[b], acc_sh.at[di_scat[b]], sems[3 * GRP - 2 + b], add=True))
            for cp in scats:
                cp.wait()
            return 0
        with jax.named_scope("phaseB"):
            lax.fori_loop(0, SEGS // GRPB, bgroup, 0)

        plsc.subcore_barrier()

        for t in range(NS // K):
            row0 = sid * NS + t * K
            pltpu.sync_copy(acc_sh.at[pl.ds(row0, K)],
                            out_hbm.at[pl.ds(cid * NP + row0, K)])

    return sc_layer


def _make_sc_pairs(NP, H, P):
    PT = P // NTILES
    mesh = plsc.VectorSubcoreMesh(core_axis_name="c", subcore_axis_name="s")

    @functools.partial(
        pl.kernel,
        out_type=[jax.ShapeDtypeStruct((P, H), jnp.float32),
                  jax.ShapeDtypeStruct((P, H), jnp.float32)],
        mesh=mesh,
        scratch_types=[
            pltpu.VMEM((K, H), jnp.float32),
            pltpu.VMEM((K, H), jnp.float32),
            pltpu.VMEM((K,), jnp.int32),
            pltpu.VMEM((K,), jnp.int32),
            pltpu.SemaphoreType.DMA,
            pltpu.SemaphoreType.DMA,
        ],
    )
    def sc_pairs(hh_hbm, ps_hbm, pd_hbm, outa_hbm, outb_hbm,
                 rs, rd, ps_v, pd_v, s0, s1):
        cid = lax.axis_index("c")
        sid = lax.axis_index("s")
        wid = sid * 2 + cid
        base = wid * PT

        def seg_body(seg, _):
            off = base + seg * K
            pltpu.sync_copy(ps_hbm.at[pl.ds(off, K)], ps_v)
            pltpu.sync_copy(pd_hbm.at[pl.ds(off, K)], pd_v)
            c0 = pltpu.async_copy(hh_hbm.at[ps_v], rs, s0)
            c1 = pltpu.async_copy(hh_hbm.at[pd_v], rd, s1)
            c0.wait()
            pltpu.sync_copy(rs, outa_hbm.at[pl.ds(off, K)])
            c1.wait()
            pltpu.sync_copy(rd, outb_hbm.at[pl.ds(off, K)])
            return 0
        lax.fori_loop(0, PT // K, seg_body, 0)

    return sc_pairs


def _relu_sum(o_flat, NP, BN):
    H = o_flat.shape[1]

    def body(x0_ref, x1_ref, o_ref):
        o_ref[...] = jnp.maximum(x0_ref[...] + x1_ref[...], 0.0)

    nblk = NP // BN
    return pl.pallas_call(
        body,
        grid=(nblk,),
        in_specs=[
            pl.BlockSpec((BN, H), lambda i: (i, 0)),
            pl.BlockSpec((BN, H), lambda i, _n=nblk: (i + _n, 0)),
        ],
        out_specs=pl.BlockSpec((BN, H), lambda i: (i, 0)),
        out_shape=jax.ShapeDtypeStruct((NP, H), jnp.float32),
    )(o_flat, o_flat)


def _final_tc(hpa, hpb, Wc, bc, BP):
    P, H = hpa.shape
    Wc1 = Wc[:H]
    Wc2 = Wc[H:]
    bc2 = bc.reshape(1, 2)

    def body(a_ref, b_ref, w1_ref, w2_ref, bc_ref, o_ref):
        z = (jnp.dot(a_ref[...], w1_ref[...], preferred_element_type=jnp.float32)
             + jnp.dot(b_ref[...], w2_ref[...], preferred_element_type=jnp.float32)
             + bc_ref[...])
        o_ref[...] = 1.0 / (1.0 + jnp.exp(-z))

    return pl.pallas_call(
        body,
        grid=(P // BP,),
        in_specs=[
            pl.BlockSpec((BP, H), lambda i: (i, 0)),
            pl.BlockSpec((BP, H), lambda i: (i, 0)),
            pl.BlockSpec((H, 2), lambda i: (0, 0)),
            pl.BlockSpec((H, 2), lambda i: (0, 0)),
            pl.BlockSpec((1, 2), lambda i: (0, 0)),
        ],
        out_specs=pl.BlockSpec((BP, 2), lambda i: (i, 0)),
        out_shape=jax.ShapeDtypeStruct((P, 2), jnp.float32),
    )(hpa, hpb, Wc1, Wc2, bc2)


def kernel(x, edge_index, pairs, W1, a1_src, a1_dst, W2, a2_src, a2_dst, Wc, bc):
    N, D = x.shape
    H = W1.shape[1]
    E = edge_index.shape[1]
    P = pairs.shape[0]

    NP = _round_up(N, LANES * K)
    EP = _round_up(E, NTILES * K * GRP)
    ET = EP // NTILES
    BN = 2048 if NP % 2048 == 0 else LANES * K

    x_pad = jnp.zeros((NP, D), jnp.float32).at[:N, :].set(x)
    pad_node = NP - 1
    src_pad = jnp.full((EP,), pad_node, jnp.int32).at[:E].set(edge_index[0])
    dst_pad = jnp.full((EP,), pad_node, jnp.int32).at[:E].set(edge_index[1])
    A1 = jnp.stack([a1_src, a1_dst], axis=1)
    A2 = jnp.stack([a2_src, a2_dst], axis=1)
    ps = pairs[:, 0].astype(jnp.int32)
    pd = pairs[:, 1].astype(jnp.int32)

    sc_layer = _make_sc_layer(NP, H, ET)
    sc_pairs = _make_sc_pairs(NP, H, P)

    def logit_bound(esd):
        m = jnp.max(esd[:, 0]) + jnp.max(esd[:, 1])
        m = jnp.where(m > 0, m, 0.2 * m)
        return jnp.full((16,), m, jnp.float32)

    h1, esd1 = _mm_first(x_pad, W1, A1, BN)
    o1 = sc_layer(h1, esd1[:, 0], esd1[:, 1], src_pad, dst_pad, logit_bound(esd1))
    h2, esd2 = _mm_relu_sum(o1, W2, A2, NP, BN)
    o2 = sc_layer(h2, esd2[:, 0], esd2[:, 1], src_pad, dst_pad, logit_bound(esd2))
    hh = _relu_sum(o2, NP, BN)
    hpa, hpb = sc_pairs(hh, ps, pd)
    return _final_tc(hpa, hpb, Wc, bc, 2048)


# speedup vs baseline: 33.4395x; 2.2402x over previous
import functools
import jax
import jax.numpy as jnp
from jax import lax
from jax.experimental import pallas as pl
from jax.experimental.pallas import tpu as pltpu
from jax.experimental.pallas import tpu_sc as plsc

LANES = 16
NTILES = 32
K = 128


def _round_up(v, m):
    return (v + m - 1) // m * m


def _mm_first(x_pad, W, A, BN):
    NP, D = x_pad.shape
    H = W.shape[1]

    def body(x_ref, w_ref, a_ref, h_ref, e_ref):
        h = jnp.dot(x_ref[...], w_ref[...], preferred_element_type=jnp.float32)
        h_ref[...] = h
        e_ref[...] = jnp.dot(h, a_ref[...], preferred_element_type=jnp.float32)

    return pl.pallas_call(
        body,
        grid=(NP // BN,),
        in_specs=[
            pl.BlockSpec((BN, D), lambda i: (i, 0)),
            pl.BlockSpec((D, H), lambda i: (0, 0)),
            pl.BlockSpec((H, 2), lambda i: (0, 0)),
        ],
        out_specs=[
            pl.BlockSpec((BN, H), lambda i: (i, 0)),
            pl.BlockSpec((BN, 2), lambda i: (i, 0)),
        ],
        out_shape=[
            jax.ShapeDtypeStruct((NP, H), jnp.float32),
            jax.ShapeDtypeStruct((NP, 2), jnp.float32),
        ],
    )(x_pad, W, A)


def _mm_relu_sum(o_flat, W, A, NP, BN):
    H = W.shape[1]
    D = o_flat.shape[1]

    def body(x0_ref, x1_ref, w_ref, a_ref, h_ref, e_ref):
        xin = jnp.maximum(x0_ref[...] + x1_ref[...], 0.0)
        h = jnp.dot(xin, w_ref[...], preferred_element_type=jnp.float32)
        h_ref[...] = h
        e_ref[...] = jnp.dot(h, a_ref[...], preferred_element_type=jnp.float32)

    nblk = NP // BN
    return pl.pallas_call(
        body,
        grid=(nblk,),
        in_specs=[
            pl.BlockSpec((BN, D), lambda i: (i, 0)),
            pl.BlockSpec((BN, D), lambda i, _n=nblk: (i + _n, 0)),
            pl.BlockSpec((D, H), lambda i: (0, 0)),
            pl.BlockSpec((H, 2), lambda i: (0, 0)),
        ],
        out_specs=[
            pl.BlockSpec((BN, H), lambda i: (i, 0)),
            pl.BlockSpec((BN, 2), lambda i: (i, 0)),
        ],
        out_shape=[
            jax.ShapeDtypeStruct((NP, H), jnp.float32),
            jax.ShapeDtypeStruct((NP, 2), jnp.float32),
        ],
    )(o_flat, o_flat, W, A)


GRP = 4


def _make_sc_layer(NP, H, ET):
    NS = NP // LANES
    SEGS = ET // K
    NGRP = SEGS // GRP
    mesh = plsc.VectorSubcoreMesh(core_axis_name="c", subcore_axis_name="s")

    @functools.partial(
        pl.kernel,
        out_type=jax.ShapeDtypeStruct((2 * NP, H), jnp.float32),
        mesh=mesh,
        scratch_types=[
            pltpu.VMEM_SHARED((NP, H), jnp.float32),
            pltpu.VMEM_SHARED((NP,), jnp.float32),
            pltpu.VMEM_SHARED((NP,), jnp.float32),
            pltpu.VMEM_SHARED((NP,), jnp.float32),
            [pltpu.VMEM((K, H), jnp.float32) for _ in range(2)],
            pltpu.VMEM((GRP * K,), jnp.int32),
            pltpu.VMEM((GRP * K,), jnp.int32),
            [pltpu.VMEM((K,), jnp.int32) for _ in range(GRP)],
            [pltpu.VMEM((K,), jnp.float32) for _ in range(GRP)],
            [pltpu.VMEM((K,), jnp.float32) for _ in range(GRP)],
            [pltpu.VMEM((K,), jnp.float32) for _ in range(GRP)],
            pltpu.VMEM((K,), jnp.float32),
            pltpu.VMEM((16,), jnp.float32),
            [pltpu.SemaphoreType.DMA for _ in range(4 * GRP)],
        ],
    )
    def sc_layer(h_hbm, es_hbm, ed_hbm, src_hbm, dst_hbm, m_hbm, out_hbm,
                 acc_sh, denom_sh, es_sh, ed_sh, rows, si_g, di_g,
                 di_scat, ga, gb, exv, attn_v, m_v, sems):
        cid = lax.axis_index("c")
        sid = lax.axis_index("s")

        @pl.when(sid == 0)
        def _():
            pltpu.sync_copy(es_hbm, es_sh)

        @pl.when(sid == 1)
        def _():
            pltpu.sync_copy(ed_hbm, ed_sh)

        zv = jnp.zeros((LANES,), jnp.float32)

        def zbody(r, _):
            for j in range(H // LANES):
                rows[0][r, pl.ds(LANES * j, LANES)] = zv
            return 0
        lax.fori_loop(0, K, zbody, 0)

        @pl.when(sid == 2)
        def _():
            def dzbody(t, _):
                pltpu.sync_copy(rows[0].at[0], denom_sh.at[pl.ds(t * H, H)])
                return 0
            lax.fori_loop(0, NP // H, dzbody, 0)

        for t in range(NS // K):
            pltpu.sync_copy(rows[0], acc_sh.at[pl.ds(sid * NS + t * K, K)])

        pltpu.sync_copy(m_hbm, m_v)
        mvec = m_v[...]
        plsc.subcore_barrier()

        own_chunk = cid * 16 + sid
        other_chunk = (1 - cid) * 16 + sid
        GK = GRP * K

        for chunk in (other_chunk, own_chunk):
            base = chunk * ET
            scope = jax.named_scope("phaseA")
            scope.__enter__()

            def agroup(g, _, base=base):
                gsl = pl.ds(base + g * GK, GK)
                ci = pltpu.async_copy(src_hbm.at[gsl], si_g, sems[3 * GRP])
                cj = pltpu.async_copy(dst_hbm.at[gsl], di_g, sems[3 * GRP + 1])
                ci.wait()
                cj.wait()
                cps = []
                for b in range(GRP):
                    sl = pl.ds(b * K, K)
                    cps.append((
                        pltpu.async_copy(es_sh.at[si_g.at[sl]], ga[b], sems[b]),
                        pltpu.async_copy(ed_sh.at[di_g.at[sl]], gb[b], sems[GRP + b]),
                    ))
                scats = []
                for b in range(GRP):
                    cps[b][0].wait()
                    cps[b][1].wait()
                    for i in range(K // LANES):
                        sl = pl.ds(LANES * i, LANES)
                        di_scat[b][sl] = di_g[pl.ds(b * K + LANES * i, LANES)]
                        e = ga[b][sl] + gb[b][sl]
                        e = jnp.where(e > 0, e, 0.2 * e)
                        exv[b][sl] = jnp.exp(e - mvec)
                    scats.append(pltpu.async_copy(
                        exv[b], denom_sh.at[di_scat[b]], sems[2 * GRP + b], add=True))
                for cp in scats:
                    cp.wait()
                return 0
            lax.fori_loop(0, NGRP, agroup, 0)
            scope.__exit__(None, None, None)

        plsc.subcore_barrier()

        GRPB = 2
        own_base = own_chunk * ET

        def bgroup(g, _):
            gsl = pl.ds(own_base + g * GRPB * K, GRPB * K)
            ci = pltpu.async_copy(src_hbm.at[gsl], si_g.at[pl.ds(0, GRPB * K)],
                                  sems[3 * GRP])
            cj = pltpu.async_copy(dst_hbm.at[gsl], di_g.at[pl.ds(0, GRPB * K)],
                                  sems[3 * GRP + 1])
            ci.wait()
            cj.wait()
            cps = []
            for b in range(GRPB):
                sl = pl.ds(b * K, K)
                cps.append((
                    pltpu.async_copy(h_hbm.at[si_g.at[sl]], rows[b], sems[b]),
                    pltpu.async_copy(es_sh.at[si_g.at[sl]], ga[b], sems[GRP + b]),
                    pltpu.async_copy(ed_sh.at[di_g.at[sl]], gb[b], sems[2 * GRP + b]),
                    pltpu.async_copy(denom_sh.at[di_g.at[sl]], exv[b], sems[3 * GRP + 2 + b]),
                ))
            scats = []
            for b in range(GRPB):
                cps[b][1].wait()
                cps[b][2].wait()
                cps[b][3].wait()
                for i in range(K // LANES):
                    sl = pl.ds(LANES * i, LANES)
                    di_scat[b][sl] = di_g[pl.ds(b * K + LANES * i, LANES)]
                    e = ga[b][sl] + gb[b][sl]
                    e = jnp.where(e > 0, e, 0.2 * e)
                    ex = jnp.exp(e - mvec)
                    attn_v[sl] = ex / (exv[b][sl] + 1e-9)
                cps[b][0].wait()

                def sbody(q, _, b=b):
                    av = attn_v[pl.ds(q * LANES, LANES)]
                    for rl in range(LANES):
                        r = q * LANES + rl
                        a = av[rl]
                        for j in range(H // LANES):
                            sl = pl.ds(LANES * j, LANES)
                            rows[b][r, sl] = rows[b][r, sl] * a
                    return 0
                lax.fori_loop(0, K // LANES, sbody, 0)
                scats.append(pltpu.async_copy(
                    rows[b], acc_sh.at[di_scat[b]], sems[3 * GRP - 2 + b], add=True))
            for cp in scats:
                cp.wait()
            return 0
        with jax.named_scope("phaseB"):
            lax.fori_loop(0, SEGS // GRPB, bgroup, 0)

        plsc.subcore_barrier()

        for t in range(NS // K):
            row0 = sid * NS + t * K
            pltpu.sync_copy(acc_sh.at[pl.ds(row0, K)],
                            out_hbm.at[pl.ds(cid * NP + row0, K)])

    return sc_layer


def _make_sc_pairs(NP, H, P):
    PT = P // NTILES
    mesh = plsc.VectorSubcoreMesh(core_axis_name="c", subcore_axis_name="s")

    @functools.partial(
        pl.kernel,
        out_type=[jax.ShapeDtypeStruct((P, H), jnp.float32),
                  jax.ShapeDtypeStruct((P, H), jnp.float32)],
        mesh=mesh,
        scratch_types=[
            pltpu.VMEM((K, H), jnp.float32),
            pltpu.VMEM((K, H), jnp.float32),
            pltpu.VMEM((K,), jnp.int32),
            pltpu.VMEM((K,), jnp.int32),
            pltpu.SemaphoreType.DMA,
            pltpu.SemaphoreType.DMA,
        ],
    )
    def sc_pairs(hh_hbm, ps_hbm, pd_hbm, outa_hbm, outb_hbm,
                 rs, rd, ps_v, pd_v, s0, s1):
        cid = lax.axis_index("c")
        sid = lax.axis_index("s")
        wid = sid * 2 + cid
        base = wid * PT

        def seg_body(seg, _):
            off = base + seg * K
            pltpu.sync_copy(ps_hbm.at[pl.ds(off, K)], ps_v)
            pltpu.sync_copy(pd_hbm.at[pl.ds(off, K)], pd_v)
            c0 = pltpu.async_copy(hh_hbm.at[ps_v], rs, s0)
            c1 = pltpu.async_copy(hh_hbm.at[pd_v], rd, s1)
            c0.wait()
            pltpu.sync_copy(rs, outa_hbm.at[pl.ds(off, K)])
            c1.wait()
            pltpu.sync_copy(rd, outb_hbm.at[pl.ds(off, K)])
            return 0
        lax.fori_loop(0, PT // K, seg_body, 0)

    return sc_pairs


def _relu_sum(o_flat, NP, BN):
    H = o_flat.shape[1]

    def body(x0_ref, x1_ref, o_ref):
        o_ref[...] = jnp.maximum(x0_ref[...] + x1_ref[...], 0.0)

    nblk = NP // BN
    return pl.pallas_call(
        body,
        grid=(nblk,),
        in_specs=[
            pl.BlockSpec((BN, H), lambda i: (i, 0)),
            pl.BlockSpec((BN, H), lambda i, _n=nblk: (i + _n, 0)),
        ],
        out_specs=pl.BlockSpec((BN, H), lambda i: (i, 0)),
        out_shape=jax.ShapeDtypeStruct((NP, H), jnp.float32),
    )(o_flat, o_flat)


def _final_tc(hpa, hpb, Wc, bc, BP):
    P, H = hpa.shape
    Wc1 = Wc[:H]
    Wc2 = Wc[H:]
    bc2 = bc.reshape(1, 2)

    def body(a_ref, b_ref, w1_ref, w2_ref, bc_ref, o_ref):
        z = (jnp.dot(a_ref[...], w1_ref[...], preferred_element_type=jnp.float32)
             + jnp.dot(b_ref[...], w2_ref[...], preferred_element_type=jnp.float32)
             + bc_ref[...])
        o_ref[...] = 1.0 / (1.0 + jnp.exp(-z))

    return pl.pallas_call(
        body,
        grid=(P // BP,),
        in_specs=[
            pl.BlockSpec((BP, H), lambda i: (i, 0)),
            pl.BlockSpec((BP, H), lambda i: (i, 0)),
            pl.BlockSpec((H, 2), lambda i: (0, 0)),
            pl.BlockSpec((H, 2), lambda i: (0, 0)),
            pl.BlockSpec((1, 2), lambda i: (0, 0)),
        ],
        out_specs=pl.BlockSpec((BP, 2), lambda i: (i, 0)),
        out_shape=jax.ShapeDtypeStruct((P, 2), jnp.float32),
    )(hpa, hpb, Wc1, Wc2, bc2)


def kernel(x, edge_index, pairs, W1, a1_src, a1_dst, W2, a2_src, a2_dst, Wc, bc):
    N, D = x.shape
    H = W1.shape[1]
    E = edge_index.shape[1]
    P = pairs.shape[0]

    NP = _round_up(N, LANES * K)
    EP = _round_up(E, NTILES * K * GRP)
    ET = EP // NTILES
    BN = 2048 if NP % 2048 == 0 else LANES * K

    x_pad = jnp.zeros((NP, D), jnp.float32).at[:N, :].set(x)
    pad_tgt = (N + jnp.arange(EP - E, dtype=jnp.int32) % (NP - N)).astype(jnp.int32)
    src_pad = jnp.concatenate([edge_index[0], pad_tgt])
    dst_pad = jnp.concatenate([edge_index[1], pad_tgt])
    A1 = jnp.stack([a1_src, a1_dst], axis=1)
    A2 = jnp.stack([a2_src, a2_dst], axis=1)
    ps = pairs[:, 0].astype(jnp.int32)
    pd = pairs[:, 1].astype(jnp.int32)

    sc_layer = _make_sc_layer(NP, H, ET)
    sc_pairs = _make_sc_pairs(NP, H, P)

    def logit_bound(esd):
        m = jnp.max(esd[:, 0]) + jnp.max(esd[:, 1])
        m = jnp.where(m > 0, m, 0.2 * m)
        return jnp.full((16,), m, jnp.float32)

    h1, esd1 = _mm_first(x_pad, W1, A1, BN)
    o1 = sc_layer(h1, esd1[:, 0], esd1[:, 1], src_pad, dst_pad, logit_bound(esd1))
    h2, esd2 = _mm_relu_sum(o1, W2, A2, NP, BN)
    o2 = sc_layer(h2, esd2[:, 0], esd2[:, 1], src_pad, dst_pad, logit_bound(esd2))
    hh = _relu_sum(o2, NP, BN)
    hpa, hpb = sc_pairs(hh, ps, pd)
    return _final_tc(hpa, hpb, Wc, bc, 2048)


# speedup vs baseline: 35.3508x; 1.0572x over previous
import functools
import jax
import jax.numpy as jnp
from jax import lax
from jax.experimental import pallas as pl
from jax.experimental.pallas import tpu as pltpu
from jax.experimental.pallas import tpu_sc as plsc

LANES = 16
NTILES = 32
K = 128


def _round_up(v, m):
    return (v + m - 1) // m * m


def _mm_first(x_pad, W, A, BN):
    NP, D = x_pad.shape
    H = W.shape[1]

    def body(x_ref, w_ref, a_ref, h_ref, e_ref):
        h = jnp.dot(x_ref[...], w_ref[...], preferred_element_type=jnp.float32)
        h_ref[...] = h
        e_ref[...] = jnp.dot(h, a_ref[...], preferred_element_type=jnp.float32)

    return pl.pallas_call(
        body,
        grid=(NP // BN,),
        in_specs=[
            pl.BlockSpec((BN, D), lambda i: (i, 0)),
            pl.BlockSpec((D, H), lambda i: (0, 0)),
            pl.BlockSpec((H, 2), lambda i: (0, 0)),
        ],
        out_specs=[
            pl.BlockSpec((BN, H), lambda i: (i, 0)),
            pl.BlockSpec((BN, 2), lambda i: (i, 0)),
        ],
        out_shape=[
            jax.ShapeDtypeStruct((NP, H), jnp.float32),
            jax.ShapeDtypeStruct((NP, 2), jnp.float32),
        ],
    )(x_pad, W, A)


def _mm_relu_sum(o_flat, W, A, NP, BN):
    H = W.shape[1]
    D = o_flat.shape[1]

    def body(x0_ref, x1_ref, w_ref, a_ref, h_ref, e_ref):
        xin = jnp.maximum(x0_ref[...] + x1_ref[...], 0.0)
        h = jnp.dot(xin, w_ref[...], preferred_element_type=jnp.float32)
        h_ref[...] = h
        e_ref[...] = jnp.dot(h, a_ref[...], preferred_element_type=jnp.float32)

    nblk = NP // BN
    return pl.pallas_call(
        body,
        grid=(nblk,),
        in_specs=[
            pl.BlockSpec((BN, D), lambda i: (i, 0)),
            pl.BlockSpec((BN, D), lambda i, _n=nblk: (i + _n, 0)),
            pl.BlockSpec((D, H), lambda i: (0, 0)),
            pl.BlockSpec((H, 2), lambda i: (0, 0)),
        ],
        out_specs=[
            pl.BlockSpec((BN, H), lambda i: (i, 0)),
            pl.BlockSpec((BN, 2), lambda i: (i, 0)),
        ],
        out_shape=[
            jax.ShapeDtypeStruct((NP, H), jnp.float32),
            jax.ShapeDtypeStruct((NP, 2), jnp.float32),
        ],
    )(o_flat, o_flat, W, A)


GRP = 8


def _make_sc_layer(NP, H, ET):
    NS = NP // LANES
    SEGS = ET // K
    NGRP = SEGS // GRP
    mesh = plsc.VectorSubcoreMesh(core_axis_name="c", subcore_axis_name="s")

    @functools.partial(
        pl.kernel,
        out_type=jax.ShapeDtypeStruct((2 * NP, H), jnp.float32),
        mesh=mesh,
        scratch_types=[
            pltpu.VMEM_SHARED((NP, H), jnp.float32),
            pltpu.VMEM_SHARED((NP,), jnp.float32),
            pltpu.VMEM_SHARED((NP,), jnp.float32),
            pltpu.VMEM_SHARED((NP,), jnp.float32),
            [pltpu.VMEM((K, H), jnp.float32) for _ in range(2)],
            pltpu.VMEM((GRP * K,), jnp.int32),
            pltpu.VMEM((GRP * K,), jnp.int32),
            [pltpu.VMEM((K,), jnp.int32) for _ in range(GRP)],
            [pltpu.VMEM((K,), jnp.float32) for _ in range(GRP)],
            [pltpu.VMEM((K,), jnp.float32) for _ in range(GRP)],
            [pltpu.VMEM((K,), jnp.float32) for _ in range(GRP)],
            pltpu.VMEM((K,), jnp.float32),
            pltpu.VMEM((16,), jnp.float32),
            [pltpu.SemaphoreType.DMA for _ in range(4 * GRP)],
        ],
    )
    def sc_layer(h_hbm, es_hbm, ed_hbm, src_hbm, dst_hbm, m_hbm, out_hbm,
                 acc_sh, denom_sh, es_sh, ed_sh, rows, si_g, di_g,
                 di_scat, ga, gb, exv, attn_v, m_v, sems):
        cid = lax.axis_index("c")
        sid = lax.axis_index("s")

        @pl.when(sid == 0)
        def _():
            pltpu.sync_copy(es_hbm, es_sh)

        @pl.when(sid == 1)
        def _():
            pltpu.sync_copy(ed_hbm, ed_sh)

        zv = jnp.zeros((LANES,), jnp.float32)

        def zbody(r, _):
            for j in range(H // LANES):
                rows[0][r, pl.ds(LANES * j, LANES)] = zv
            return 0
        lax.fori_loop(0, K, zbody, 0)

        for t in range(NS // H):
            pltpu.sync_copy(rows[0].at[0], denom_sh.at[pl.ds(sid * NS + t * H, H)])

        for t in range(NS // K):
            pltpu.sync_copy(rows[0], acc_sh.at[pl.ds(sid * NS + t * K, K)])

        pltpu.sync_copy(m_hbm, m_v)
        mvec = m_v[...]
        plsc.subcore_barrier()

        own_chunk = cid * 16 + sid
        other_chunk = (1 - cid) * 16 + sid
        GK = GRP * K

        for chunk in (other_chunk, own_chunk):
            base = chunk * ET
            scope = jax.named_scope("phaseA")
            scope.__enter__()

            def agroup(g, _, base=base):
                gsl = pl.ds(base + g * GK, GK)
                ci = pltpu.async_copy(src_hbm.at[gsl], si_g, sems[3 * GRP])
                cj = pltpu.async_copy(dst_hbm.at[gsl], di_g, sems[3 * GRP + 1])
                ci.wait()
                cj.wait()
                cps = []
                for b in range(GRP):
                    sl = pl.ds(b * K, K)
                    cps.append((
                        pltpu.async_copy(es_sh.at[si_g.at[sl]], ga[b], sems[b]),
                        pltpu.async_copy(ed_sh.at[di_g.at[sl]], gb[b], sems[GRP + b]),
                    ))
                scats = []
                for b in range(GRP):
                    cps[b][0].wait()
                    cps[b][1].wait()
                    for i in range(K // LANES):
                        sl = pl.ds(LANES * i, LANES)
                        di_scat[b][sl] = di_g[pl.ds(b * K + LANES * i, LANES)]
                        e = ga[b][sl] + gb[b][sl]
                        e = jnp.where(e > 0, e, 0.2 * e)
                        exv[b][sl] = jnp.exp(e - mvec)
                    scats.append(pltpu.async_copy(
                        exv[b], denom_sh.at[di_scat[b]], sems[2 * GRP + b], add=True))
                for cp in scats:
                    cp.wait()
                return 0
            lax.fori_loop(0, NGRP, agroup, 0)
            scope.__exit__(None, None, None)

        plsc.subcore_barrier()

        GRPB = 2
        own_base = own_chunk * ET

        def bgroup(g, _):
            gsl = pl.ds(own_base + g * GRPB * K, GRPB * K)
            ci = pltpu.async_copy(src_hbm.at[gsl], si_g.at[pl.ds(0, GRPB * K)],
                                  sems[3 * GRP])
            cj = pltpu.async_copy(dst_hbm.at[gsl], di_g.at[pl.ds(0, GRPB * K)],
                                  sems[3 * GRP + 1])
            ci.wait()
            cj.wait()
            cps = []
            for b in range(GRPB):
                sl = pl.ds(b * K, K)
                cps.append((
                    pltpu.async_copy(h_hbm.at[si_g.at[sl]], rows[b], sems[b]),
                    pltpu.async_copy(es_sh.at[si_g.at[sl]], ga[b], sems[GRP + b]),
                    pltpu.async_copy(ed_sh.at[di_g.at[sl]], gb[b], sems[2 * GRP + b]),
                    pltpu.async_copy(denom_sh.at[di_g.at[sl]], exv[b], sems[3 * GRP + 2 + b]),
                ))
            scats = []
            for b in range(GRPB):
                cps[b][1].wait()
                cps[b][2].wait()
                cps[b][3].wait()
                for i in range(K // LANES):
                    sl = pl.ds(LANES * i, LANES)
                    di_scat[b][sl] = di_g[pl.ds(b * K + LANES * i, LANES)]
                    e = ga[b][sl] + gb[b][sl]
                    e = jnp.where(e > 0, e, 0.2 * e)
                    ex = jnp.exp(e - mvec)
                    attn_v[sl] = ex / (exv[b][sl] + 1e-9)
                cps[b][0].wait()

                def sbody(q, _, b=b):
                    av = attn_v[pl.ds(q * LANES, LANES)]
                    for rl in range(LANES):
                        r = q * LANES + rl
                        a = av[rl]
                        for j in range(H // LANES):
                            sl = pl.ds(LANES * j, LANES)
                            rows[b][r, sl] = rows[b][r, sl] * a
                    return 0
                lax.fori_loop(0, K // LANES, sbody, 0)
                scats.append(pltpu.async_copy(
                    rows[b], acc_sh.at[di_scat[b]], sems[3 * GRP - 2 + b], add=True))
            for cp in scats:
                cp.wait()
            return 0
        with jax.named_scope("phaseB"):
            lax.fori_loop(0, SEGS // GRPB, bgroup, 0)

        plsc.subcore_barrier()

        for t in range(NS // K):
            row0 = sid * NS + t * K
            pltpu.sync_copy(acc_sh.at[pl.ds(row0, K)],
                            out_hbm.at[pl.ds(cid * NP + row0, K)])

    return sc_layer


def _make_sc_pairs(NP, H, P):
    PT = P // NTILES
    mesh = plsc.VectorSubcoreMesh(core_axis_name="c", subcore_axis_name="s")

    @functools.partial(
        pl.kernel,
        out_type=[jax.ShapeDtypeStruct((P, H), jnp.float32),
                  jax.ShapeDtypeStruct((P, H), jnp.float32)],
        mesh=mesh,
        scratch_types=[
            pltpu.VMEM((K, H), jnp.float32),
            pltpu.VMEM((K, H), jnp.float32),
            pltpu.VMEM((K,), jnp.int32),
            pltpu.VMEM((K,), jnp.int32),
            pltpu.SemaphoreType.DMA,
            pltpu.SemaphoreType.DMA,
        ],
    )
    def sc_pairs(hh_hbm, ps_hbm, pd_hbm, outa_hbm, outb_hbm,
                 rs, rd, ps_v, pd_v, s0, s1):
        cid = lax.axis_index("c")
        sid = lax.axis_index("s")
        wid = sid * 2 + cid
        base = wid * PT

        def seg_body(seg, _):
            off = base + seg * K
            pltpu.sync_copy(ps_hbm.at[pl.ds(off, K)], ps_v)
            pltpu.sync_copy(pd_hbm.at[pl.ds(off, K)], pd_v)
            c0 = pltpu.async_copy(hh_hbm.at[ps_v], rs, s0)
            c1 = pltpu.async_copy(hh_hbm.at[pd_v], rd, s1)
            c0.wait()
            pltpu.sync_copy(rs, outa_hbm.at[pl.ds(off, K)])
            c1.wait()
            pltpu.sync_copy(rd, outb_hbm.at[pl.ds(off, K)])
            return 0
        lax.fori_loop(0, PT // K, seg_body, 0)

    return sc_pairs


def _relu_sum(o_flat, NP, BN):
    H = o_flat.shape[1]

    def body(x0_ref, x1_ref, o_ref):
        o_ref[...] = jnp.maximum(x0_ref[...] + x1_ref[...], 0.0)

    nblk = NP // BN
    return pl.pallas_call(
        body,
        grid=(nblk,),
        in_specs=[
            pl.BlockSpec((BN, H), lambda i: (i, 0)),
            pl.BlockSpec((BN, H), lambda i, _n=nblk: (i + _n, 0)),
        ],
        out_specs=pl.BlockSpec((BN, H), lambda i: (i, 0)),
        out_shape=jax.ShapeDtypeStruct((NP, H), jnp.float32),
    )(o_flat, o_flat)


def _final_tc(hpa, hpb, Wc, bc, BP):
    P, H = hpa.shape
    Wc1 = Wc[:H]
    Wc2 = Wc[H:]
    bc2 = bc.reshape(1, 2)

    def body(a_ref, b_ref, w1_ref, w2_ref, bc_ref, o_ref):
        z = (jnp.dot(a_ref[...], w1_ref[...], preferred_element_type=jnp.float32)
             + jnp.dot(b_ref[...], w2_ref[...], preferred_element_type=jnp.float32)
             + bc_ref[...])
        o_ref[...] = 1.0 / (1.0 + jnp.exp(-z))

    return pl.pallas_call(
        body,
        grid=(P // BP,),
        in_specs=[
            pl.BlockSpec((BP, H), lambda i: (i, 0)),
            pl.BlockSpec((BP, H), lambda i: (i, 0)),
            pl.BlockSpec((H, 2), lambda i: (0, 0)),
            pl.BlockSpec((H, 2), lambda i: (0, 0)),
            pl.BlockSpec((1, 2), lambda i: (0, 0)),
        ],
        out_specs=pl.BlockSpec((BP, 2), lambda i: (i, 0)),
        out_shape=jax.ShapeDtypeStruct((P, 2), jnp.float32),
    )(hpa, hpb, Wc1, Wc2, bc2)


def kernel(x, edge_index, pairs, W1, a1_src, a1_dst, W2, a2_src, a2_dst, Wc, bc):
    N, D = x.shape
    H = W1.shape[1]
    E = edge_index.shape[1]
    P = pairs.shape[0]

    NP = _round_up(N, LANES * K)
    EP = _round_up(E, NTILES * K * GRP)
    ET = EP // NTILES
    BN = 2048 if NP % 2048 == 0 else LANES * K

    x_pad = jnp.zeros((NP, D), jnp.float32).at[:N, :].set(x)
    pad_tgt = (N + jnp.arange(EP - E, dtype=jnp.int32) % (NP - N)).astype(jnp.int32)
    src_pad = jnp.concatenate([edge_index[0], pad_tgt])
    dst_pad = jnp.concatenate([edge_index[1], pad_tgt])
    A1 = jnp.stack([a1_src, a1_dst], axis=1)
    A2 = jnp.stack([a2_src, a2_dst], axis=1)
    ps = pairs[:, 0].astype(jnp.int32)
    pd = pairs[:, 1].astype(jnp.int32)

    sc_layer = _make_sc_layer(NP, H, ET)
    sc_pairs = _make_sc_pairs(NP, H, P)

    def logit_bound(esd):
        m = jnp.max(esd[:, 0]) + jnp.max(esd[:, 1])
        m = jnp.where(m > 0, m, 0.2 * m)
        return jnp.full((16,), m, jnp.float32)

    h1, esd1 = _mm_first(x_pad, W1, A1, BN)
    o1 = sc_layer(h1, esd1[:, 0], esd1[:, 1], src_pad, dst_pad, logit_bound(esd1))
    h2, esd2 = _mm_relu_sum(o1, W2, A2, NP, BN)
    o2 = sc_layer(h2, esd2[:, 0], esd2[:, 1], src_pad, dst_pad, logit_bound(esd2))
    hh = _relu_sum(o2, NP, BN)
    hpa, hpb = sc_pairs(hh, ps, pd)
    return _final_tc(hpa, hpb, Wc, bc, 2048)


# speedup vs baseline: 36.5343x; 1.0335x over previous
import functools
import jax
import jax.numpy as jnp
from jax import lax
from jax.experimental import pallas as pl
from jax.experimental.pallas import tpu as pltpu
from jax.experimental.pallas import tpu_sc as plsc

LANES = 16
NTILES = 32
K = 128


def _round_up(v, m):
    return (v + m - 1) // m * m


def _mm_first(x_pad, W, A, BN):
    NP, D = x_pad.shape
    H = W.shape[1]

    def body(x_ref, w_ref, a_ref, h_ref, e_ref):
        h = jnp.dot(x_ref[...], w_ref[...], preferred_element_type=jnp.float32)
        h_ref[...] = h
        e_ref[...] = jnp.dot(h, a_ref[...], preferred_element_type=jnp.float32)

    return pl.pallas_call(
        body,
        grid=(NP // BN,),
        in_specs=[
            pl.BlockSpec((BN, D), lambda i: (i, 0)),
            pl.BlockSpec((D, H), lambda i: (0, 0)),
            pl.BlockSpec((H, 2), lambda i: (0, 0)),
        ],
        out_specs=[
            pl.BlockSpec((BN, H), lambda i: (i, 0)),
            pl.BlockSpec((BN, 2), lambda i: (i, 0)),
        ],
        out_shape=[
            jax.ShapeDtypeStruct((NP, H), jnp.float32),
            jax.ShapeDtypeStruct((NP, 2), jnp.float32),
        ],
    )(x_pad, W, A)


def _mm_relu_sum(o_flat, W, A, NP, BN):
    H = W.shape[1]
    D = o_flat.shape[1]

    def body(x0_ref, x1_ref, w_ref, a_ref, h_ref, e_ref):
        xin = jnp.maximum(x0_ref[...] + x1_ref[...], 0.0)
        h = jnp.dot(xin, w_ref[...], preferred_element_type=jnp.float32)
        h_ref[...] = h
        e_ref[...] = jnp.dot(h, a_ref[...], preferred_element_type=jnp.float32)

    nblk = NP // BN
    return pl.pallas_call(
        body,
        grid=(nblk,),
        in_specs=[
            pl.BlockSpec((BN, D), lambda i: (i, 0)),
            pl.BlockSpec((BN, D), lambda i, _n=nblk: (i + _n, 0)),
            pl.BlockSpec((D, H), lambda i: (0, 0)),
            pl.BlockSpec((H, 2), lambda i: (0, 0)),
        ],
        out_specs=[
            pl.BlockSpec((BN, H), lambda i: (i, 0)),
            pl.BlockSpec((BN, 2), lambda i: (i, 0)),
        ],
        out_shape=[
            jax.ShapeDtypeStruct((NP, H), jnp.float32),
            jax.ShapeDtypeStruct((NP, 2), jnp.float32),
        ],
    )(o_flat, o_flat, W, A)


GRP = 8


def _make_sc_layer(NP, H, ET):
    NS = NP // LANES
    SEGS = ET // K
    NGRP = SEGS // GRP
    mesh = plsc.VectorSubcoreMesh(core_axis_name="c", subcore_axis_name="s")

    @functools.partial(
        pl.kernel,
        out_type=jax.ShapeDtypeStruct((2 * NP, H), jnp.float32),
        mesh=mesh,
        scratch_types=[
            pltpu.VMEM_SHARED((NP, H), jnp.float32),
            pltpu.VMEM_SHARED((NP,), jnp.float32),
            pltpu.VMEM_SHARED((NP,), jnp.float32),
            pltpu.VMEM_SHARED((NP,), jnp.float32),
            [pltpu.VMEM((K, H), jnp.float32) for _ in range(2)],
            pltpu.VMEM((GRP * K,), jnp.int32),
            pltpu.VMEM((GRP * K,), jnp.int32),
            [pltpu.VMEM((K,), jnp.int32) for _ in range(GRP)],
            [pltpu.VMEM((K,), jnp.float32) for _ in range(GRP)],
            [pltpu.VMEM((K,), jnp.float32) for _ in range(GRP)],
            [pltpu.VMEM((K,), jnp.float32) for _ in range(GRP)],
            pltpu.VMEM((K,), jnp.float32),
            pltpu.VMEM((16,), jnp.float32),
            [pltpu.SemaphoreType.DMA for _ in range(4 * GRP)],
        ],
    )
    def sc_layer(h_hbm, es_hbm, ed_hbm, src_hbm, dst_hbm, m_hbm, out_hbm,
                 acc_sh, denom_sh, es_sh, ed_sh, rows, si_g, di_g,
                 di_scat, ga, gb, exv, attn_v, m_v, sems):
        cid = lax.axis_index("c")
        sid = lax.axis_index("s")

        @pl.when(sid == 0)
        def _():
            pltpu.sync_copy(es_hbm, es_sh)

        @pl.when(sid == 1)
        def _():
            pltpu.sync_copy(ed_hbm, ed_sh)

        zv = jnp.zeros((LANES,), jnp.float32)

        def zbody(r, _):
            for j in range(H // LANES):
                rows[0][r, pl.ds(LANES * j, LANES)] = zv
            return 0
        lax.fori_loop(0, K, zbody, 0)

        for t in range(NS // H):
            pltpu.sync_copy(rows[0].at[0], denom_sh.at[pl.ds(sid * NS + t * H, H)])

        for t in range(NS // K):
            pltpu.sync_copy(rows[0], acc_sh.at[pl.ds(sid * NS + t * K, K)])

        pltpu.sync_copy(m_hbm, m_v)
        mvec = m_v[...]
        plsc.subcore_barrier()

        own_chunk = cid * 16 + sid
        other_chunk = (1 - cid) * 16 + sid
        GK = GRP * K

        for chunk in (other_chunk, own_chunk):
            base = chunk * ET
            scope = jax.named_scope("phaseA")
            scope.__enter__()

            def agroup(g, _, base=base):
                gsl = pl.ds(base + g * GK, GK)
                ci = pltpu.async_copy(src_hbm.at[gsl], si_g, sems[3 * GRP])
                cj = pltpu.async_copy(dst_hbm.at[gsl], di_g, sems[3 * GRP + 1])
                ci.wait()
                cj.wait()
                cps = []
                for b in range(GRP):
                    sl = pl.ds(b * K, K)
                    cps.append((
                        pltpu.async_copy(es_sh.at[si_g.at[sl]], ga[b], sems[b]),
                        pltpu.async_copy(ed_sh.at[di_g.at[sl]], gb[b], sems[GRP + b]),
                    ))
                scats = []
                for b in range(GRP):
                    cps[b][0].wait()
                    cps[b][1].wait()
                    for i in range(K // LANES):
                        sl = pl.ds(LANES * i, LANES)
                        di_scat[b][sl] = di_g[pl.ds(b * K + LANES * i, LANES)]
                        e = ga[b][sl] + gb[b][sl]
                        e = jnp.where(e > 0, e, 0.2 * e)
                        exv[b][sl] = jnp.exp(e - mvec)
                    scats.append(pltpu.async_copy(
                        exv[b], denom_sh.at[di_scat[b]], sems[2 * GRP + b], add=True))
                for cp in scats:
                    cp.wait()
                return 0
            lax.fori_loop(0, NGRP, agroup, 0)
            scope.__exit__(None, None, None)

        plsc.subcore_barrier()

        GRPB = 2
        own_base = own_chunk * ET

        def bgroup(g, _):
            gsl = pl.ds(own_base + g * GK, GK)
            ci = pltpu.async_copy(src_hbm.at[gsl], si_g, sems[3 * GRP])
            cj = pltpu.async_copy(dst_hbm.at[gsl], di_g, sems[3 * GRP + 1])
            ci.wait()
            cj.wait()
            for sub in range(GRP // GRPB):
                cps = []
                for b in range(GRPB):
                    sloc = sub * GRPB + b
                    sl = pl.ds(sloc * K, K)
                    cps.append((
                        pltpu.async_copy(h_hbm.at[si_g.at[sl]], rows[b], sems[b]),
                        pltpu.async_copy(es_sh.at[si_g.at[sl]], ga[b], sems[GRP + b]),
                        pltpu.async_copy(ed_sh.at[di_g.at[sl]], gb[b], sems[2 * GRP + b]),
                        pltpu.async_copy(denom_sh.at[di_g.at[sl]], exv[b], sems[3 * GRP + 2 + b]),
                    ))
                scats = []
                for b in range(GRPB):
                    sloc = sub * GRPB + b
                    cps[b][1].wait()
                    cps[b][2].wait()
                    cps[b][3].wait()
                    for i in range(K // LANES):
                        sl = pl.ds(LANES * i, LANES)
                        di_scat[b][sl] = di_g[pl.ds(sloc * K + LANES * i, LANES)]
                        e = ga[b][sl] + gb[b][sl]
                        e = jnp.where(e > 0, e, 0.2 * e)
                        ex = jnp.exp(e - mvec)
                        attn_v[sl] = ex / (exv[b][sl] + 1e-9)
                    cps[b][0].wait()

                    def sbody(q, _, b=b):
                        av = attn_v[pl.ds(q * LANES, LANES)]
                        for rl in range(LANES):
                            r = q * LANES + rl
                            a = av[rl]
                            for j in range(H // LANES):
                                sl = pl.ds(LANES * j, LANES)
                                rows[b][r, sl] = rows[b][r, sl] * a
                        return 0
                    lax.fori_loop(0, K // LANES, sbody, 0)
                    scats.append(pltpu.async_copy(
                        rows[b], acc_sh.at[di_scat[b]], sems[3 * GRP - 2 + b], add=True))
                for cp in scats:
                    cp.wait()
            return 0
        with jax.named_scope("phaseB"):
            lax.fori_loop(0, SEGS // GRP, bgroup, 0)

        plsc.subcore_barrier()

        for t in range(NS // K):
            row0 = sid * NS + t * K
            pltpu.sync_copy(acc_sh.at[pl.ds(row0, K)],
                            out_hbm.at[pl.ds(cid * NP + row0, K)])

    return sc_layer


def _make_sc_pairs(NP, H, P):
    PT = P // NTILES
    mesh = plsc.VectorSubcoreMesh(core_axis_name="c", subcore_axis_name="s")

    @functools.partial(
        pl.kernel,
        out_type=[jax.ShapeDtypeStruct((P, H), jnp.float32),
                  jax.ShapeDtypeStruct((P, H), jnp.float32)],
        mesh=mesh,
        scratch_types=[
            pltpu.VMEM((K, H), jnp.float32),
            pltpu.VMEM((K, H), jnp.float32),
            pltpu.VMEM((K,), jnp.int32),
            pltpu.VMEM((K,), jnp.int32),
            pltpu.SemaphoreType.DMA,
            pltpu.SemaphoreType.DMA,
        ],
    )
    def sc_pairs(hh_hbm, ps_hbm, pd_hbm, outa_hbm, outb_hbm,
                 rs, rd, ps_v, pd_v, s0, s1):
        cid = lax.axis_index("c")
        sid = lax.axis_index("s")
        wid = sid * 2 + cid
        base = wid * PT

        def seg_body(seg, _):
            off = base + seg * K
            pltpu.sync_copy(ps_hbm.at[pl.ds(off, K)], ps_v)
            pltpu.sync_copy(pd_hbm.at[pl.ds(off, K)], pd_v)
            c0 = pltpu.async_copy(hh_hbm.at[ps_v], rs, s0)
            c1 = pltpu.async_copy(hh_hbm.at[pd_v], rd, s1)
            c0.wait()
            pltpu.sync_copy(rs, outa_hbm.at[pl.ds(off, K)])
            c1.wait()
            pltpu.sync_copy(rd, outb_hbm.at[pl.ds(off, K)])
            return 0
        lax.fori_loop(0, PT // K, seg_body, 0)

    return sc_pairs


def _relu_sum(o_flat, NP, BN):
    H = o_flat.shape[1]

    def body(x0_ref, x1_ref, o_ref):
        o_ref[...] = jnp.maximum(x0_ref[...] + x1_ref[...], 0.0)

    nblk = NP // BN
    return pl.pallas_call(
        body,
        grid=(nblk,),
        in_specs=[
            pl.BlockSpec((BN, H), lambda i: (i, 0)),
            pl.BlockSpec((BN, H), lambda i, _n=nblk: (i + _n, 0)),
        ],
        out_specs=pl.BlockSpec((BN, H), lambda i: (i, 0)),
        out_shape=jax.ShapeDtypeStruct((NP, H), jnp.float32),
    )(o_flat, o_flat)


def _final_tc(hpa, hpb, Wc, bc, BP):
    P, H = hpa.shape
    Wc1 = Wc[:H]
    Wc2 = Wc[H:]
    bc2 = bc.reshape(1, 2)

    def body(a_ref, b_ref, w1_ref, w2_ref, bc_ref, o_ref):
        z = (jnp.dot(a_ref[...], w1_ref[...], preferred_element_type=jnp.float32)
             + jnp.dot(b_ref[...], w2_ref[...], preferred_element_type=jnp.float32)
             + bc_ref[...])
        o_ref[...] = 1.0 / (1.0 + jnp.exp(-z))

    return pl.pallas_call(
        body,
        grid=(P // BP,),
        in_specs=[
            pl.BlockSpec((BP, H), lambda i: (i, 0)),
            pl.BlockSpec((BP, H), lambda i: (i, 0)),
            pl.BlockSpec((H, 2), lambda i: (0, 0)),
            pl.BlockSpec((H, 2), lambda i: (0, 0)),
            pl.BlockSpec((1, 2), lambda i: (0, 0)),
        ],
        out_specs=pl.BlockSpec((BP, 2), lambda i: (i, 0)),
        out_shape=jax.ShapeDtypeStruct((P, 2), jnp.float32),
    )(hpa, hpb, Wc1, Wc2, bc2)


def kernel(x, edge_index, pairs, W1, a1_src, a1_dst, W2, a2_src, a2_dst, Wc, bc):
    N, D = x.shape
    H = W1.shape[1]
    E = edge_index.shape[1]
    P = pairs.shape[0]

    NP = _round_up(N, LANES * K)
    EP = _round_up(E, NTILES * K * GRP)
    ET = EP // NTILES
    BN = 2048 if NP % 2048 == 0 else LANES * K

    x_pad = jnp.zeros((NP, D), jnp.float32).at[:N, :].set(x)
    pad_tgt = (N + jnp.arange(EP - E, dtype=jnp.int32) % (NP - N)).astype(jnp.int32)
    src_pad = jnp.concatenate([edge_index[0], pad_tgt])
    dst_pad = jnp.concatenate([edge_index[1], pad_tgt])
    A1 = jnp.stack([a1_src, a1_dst], axis=1)
    A2 = jnp.stack([a2_src, a2_dst], axis=1)
    ps = pairs[:, 0].astype(jnp.int32)
    pd = pairs[:, 1].astype(jnp.int32)

    sc_layer = _make_sc_layer(NP, H, ET)
    sc_pairs = _make_sc_pairs(NP, H, P)

    def logit_bound(esd):
        m = jnp.max(esd[:, 0]) + jnp.max(esd[:, 1])
        m = jnp.where(m > 0, m, 0.2 * m)
        return jnp.full((16,), m, jnp.float32)

    h1, esd1 = _mm_first(x_pad, W1, A1, BN)
    o1 = sc_layer(h1, esd1[:, 0], esd1[:, 1], src_pad, dst_pad, logit_bound(esd1))
    h2, esd2 = _mm_relu_sum(o1, W2, A2, NP, BN)
    o2 = sc_layer(h2, esd2[:, 0], esd2[:, 1], src_pad, dst_pad, logit_bound(esd2))
    hh = _relu_sum(o2, NP, BN)
    hpa, hpb = sc_pairs(hh, ps, pd)
    return _final_tc(hpa, hpb, Wc, bc, 2048)
